# baseline (speedup 1.0000x reference)
import functools

import jax
import jax.numpy as jnp
from jax import lax
from jax.experimental import pallas as pl
from jax.experimental.pallas import tpu as pltpu
from jax.experimental.pallas import tpu_sc as plsc

B, C_IN, H, W = 1, 2048, 15, 20
FEAT = 512
SCALE = 16
OH, OW = H * SCALE, W * SCALE
HW = OH * OW
NPIX = H * W
NXUP = H * OW
N_VOX = 60 * 36 * 60

NW = 32
NZ = 60
VB = N_VOX // NZ
NBLK = NZ
CPT = 8
NCH = FEAT // (NW * CPT)


def _stage1_body(w_ref, f2d_ref, gamma_ref, beta_ref, out_ref):
    x = jnp.dot(w_ref[...], f2d_ref[...], preferred_element_type=jnp.float32)
    mean = jnp.mean(x, axis=1, keepdims=True)
    var = jnp.mean(x * x, axis=1, keepdims=True) - mean * mean
    x = (x - mean) * jax.lax.rsqrt(var + 1e-5)
    x = x * gamma_ref[...] + beta_ref[...]
    out_ref[...] = jnp.maximum(x, 0.0)


def _xup_body(tbl_ref, out_ref):
    oxi = lax.broadcasted_iota(jnp.int32, (W, OW), 1)
    j = lax.broadcasted_iota(jnp.int32, (W, OW), 0)
    fx = oxi.astype(jnp.float32) * (float(W - 1) / (OW - 1))
    x0 = jnp.floor(fx)
    dx = fx - x0
    x0i = x0.astype(jnp.int32)
    x1i = jnp.minimum(x0i + 1, W - 1)
    wx = (jnp.where(j == x0i, 1.0 - dx, 0.0)
          + jnp.where(j == x1i, dx, 0.0))
    for y in range(H):
        out_ref[:, y * OW:(y + 1) * OW] = jnp.dot(
            tbl_ref[:, y * W:(y + 1) * W], wx,
            preferred_element_type=jnp.float32)


def _prep_body(idx_ref, c0_ref, c1_ref, w0_ref, w1_ref):
    v = idx_ref[...]
    valid = v < HW
    vc = jnp.where(valid, v, 0)
    py = vc // OW
    px = vc - py * OW
    fy = py.astype(jnp.float32) * (float(H - 1) / (OH - 1))
    y0 = jnp.floor(fy)
    dy = fy - y0
    y0i = y0.astype(jnp.int32)
    y1i = jnp.minimum(y0i + 1, H - 1)
    vf = jnp.where(valid, 1.0, 0.0)
    c0_ref[...] = y0i * OW + px
    c1_ref[...] = y1i * OW + px
    w0_ref[...] = (1.0 - dy) * vf
    w1_ref[...] = dy * vf


def _sc_gather(tblx_hbm, c0_h, c1_h, w0_h, w1_h, out_hbm,
               tbl_v, ci_v, wf_v, out_v, sem_in, sem_out):
    wid = lax.axis_index("s") * 2 + lax.axis_index("c")

    def in_copies(b, par):
        vbase = b * VB
        return [
            pltpu.make_async_copy(
                c0_h.at[pl.ds(vbase, VB)], ci_v.at[par, 0], sem_in),
            pltpu.make_async_copy(
                c1_h.at[pl.ds(vbase, VB)], ci_v.at[par, 1], sem_in),
            pltpu.make_async_copy(
                w0_h.at[pl.ds(vbase, VB)], wf_v.at[par, 0], sem_in),
            pltpu.make_async_copy(
                w1_h.at[pl.ds(vbase, VB)], wf_v.at[par, 1], sem_in),
        ]

    def out_copy(rows, b, par):
        return pltpu.make_async_copy(
            out_v.at[par],
            out_hbm.at[pl.ds(rows, CPT), b], sem_out)

    for h in range(NCH):
        rows = h * (NW * CPT) + wid * CPT
        pltpu.sync_copy(tblx_hbm.at[pl.ds(rows, CPT), :], tbl_v)
        for cp in in_copies(0, 0):
            cp.start()

        def blk_body(b, carry):
            par = lax.rem(b, 2)
            for cp in in_copies(b, par):
                cp.wait()

            @pl.when(b + 1 < NBLK)
            def _():
                for cp in in_copies(b + 1, 1 - par):
                    cp.start()

            @pl.when(b >= 2)
            def _():
                out_copy(rows, b - 2, par).wait()

            def g_body(g, carry2):
                s = pl.ds(g * 16, 16)
                c0 = ci_v[par, 0, s]
                c1 = ci_v[par, 1, s]
                w0 = wf_v[par, 0, s]
                w1 = wf_v[par, 1, s]
                accs = []
                for c in range(CPT):
                    cvec = jnp.full((16,), c, jnp.int32)
                    accs.append(w0 * plsc.load_gather(tbl_v, [cvec, c0])
                                + w1 * plsc.load_gather(tbl_v, [cvec, c1]))
                for c in range(CPT):
                    out_v[par, c, s] = accs[c]
                return carry2

            lax.fori_loop(0, VB // 16, g_body, 0)
            out_copy(rows, b, par).start()
            return carry

        lax.fori_loop(0, NBLK, blk_body, 0)
        out_copy(rows, NBLK - 2, lax.rem(NBLK - 2, 2)).wait()
        out_copy(rows, NBLK - 1, lax.rem(NBLK - 1, 2)).wait()


def kernel(feature2d, depth_mapping_3d, conv_w, bn_gamma, bn_beta):
    f2d = feature2d.reshape(C_IN, NPIX)
    idx = depth_mapping_3d.reshape(N_VOX).astype(jnp.int32)
    tbl = pl.pallas_call(
        _stage1_body,
        out_shape=jax.ShapeDtypeStruct((FEAT, NPIX), jnp.float32),
    )(conv_w, f2d, bn_gamma.reshape(FEAT, 1), bn_beta.reshape(FEAT, 1))

    tblx = pl.pallas_call(
        _xup_body,
        out_shape=jax.ShapeDtypeStruct((FEAT, NXUP), jnp.float32),
    )(tbl)

    c0, c1, w0, w1 = pl.pallas_call(
        _prep_body,
        out_shape=[jax.ShapeDtypeStruct((N_VOX,), jnp.int32)] * 2
        + [jax.ShapeDtypeStruct((N_VOX,), jnp.float32)] * 2,
    )(idx)

    mesh = plsc.VectorSubcoreMesh(core_axis_name="c", subcore_axis_name="s")
    sc = functools.partial(
        pl.kernel,
        mesh=mesh,
        out_type=jax.ShapeDtypeStruct((FEAT, NZ, VB), jnp.float32),
        scratch_types=[
            pltpu.VMEM((CPT, NXUP), jnp.float32),
            pltpu.VMEM((2, 2, VB), jnp.int32),
            pltpu.VMEM((2, 2, VB), jnp.float32),
            pltpu.VMEM((2, CPT, VB), jnp.float32),
            pltpu.SemaphoreType.DMA,
            pltpu.SemaphoreType.DMA,
        ],
        compiler_params=pltpu.CompilerParams(
            use_tc_tiling_on_sc=False, needs_layout_passes=False),
    )(_sc_gather)
    out = sc(tblx, c0, c1, w0, w1)
    return out.reshape(1, FEAT, 60, 36, 60)


# speedup vs baseline: 1.0633x; 1.0633x over previous
import functools

import jax
import jax.numpy as jnp
from jax import lax
from jax.experimental import pallas as pl
from jax.experimental.pallas import tpu as pltpu
from jax.experimental.pallas import tpu_sc as plsc

B, C_IN, H, W = 1, 2048, 15, 20
FEAT = 512
SCALE = 16
OH, OW = H * SCALE, W * SCALE
HW = OH * OW
NPIX = H * W
NXUP = H * OW
N_VOX = 60 * 36 * 60

NW = 32
CPT = FEAT // NW
VB = 576
NBLK = N_VOX // VB


def _stage1_body(w_ref, f2d_ref, gamma_ref, beta_ref, out_ref):
    x = jnp.dot(w_ref[...], f2d_ref[...], preferred_element_type=jnp.float32)
    mean = jnp.mean(x, axis=1, keepdims=True)
    var = jnp.mean(x * x, axis=1, keepdims=True) - mean * mean
    x = (x - mean) * jax.lax.rsqrt(var + 1e-5)
    x = x * gamma_ref[...] + beta_ref[...]
    out_ref[...] = jnp.maximum(x, 0.0)


def _xup_body(tbl_ref, out_ref):
    oxi = lax.broadcasted_iota(jnp.int32, (W, OW), 1)
    j = lax.broadcasted_iota(jnp.int32, (W, OW), 0)
    fx = oxi.astype(jnp.float32) * (float(W - 1) / (OW - 1))
    x0 = jnp.floor(fx)
    dx = fx - x0
    x0i = x0.astype(jnp.int32)
    x1i = jnp.minimum(x0i + 1, W - 1)
    wx = (jnp.where(j == x0i, 1.0 - dx, 0.0)
          + jnp.where(j == x1i, dx, 0.0))
    for y in range(H):
        out_ref[:, y * OW:(y + 1) * OW] = jnp.dot(
            tbl_ref[:, y * W:(y + 1) * W], wx,
            preferred_element_type=jnp.float32)


def _prep_body(idx_ref, c0_ref, c1_ref, w0_ref, w1_ref):
    v = idx_ref[...]
    valid = v < HW
    vc = jnp.where(valid, v, 0)
    py = vc // OW
    px = vc - py * OW
    fy = py.astype(jnp.float32) * (float(H - 1) / (OH - 1))
    y0 = jnp.floor(fy)
    dy = fy - y0
    y0i = y0.astype(jnp.int32)
    y1i = jnp.minimum(y0i + 1, H - 1)
    vf = jnp.where(valid, 1.0, 0.0)
    c0_ref[...] = y0i * OW + px
    c1_ref[...] = y1i * OW + px
    w0_ref[...] = (1.0 - dy) * vf
    w1_ref[...] = dy * vf


def _sc_gather(tblx_hbm, c0_h, c1_h, w0_h, w1_h, out_hbm,
               tbl_v, ci_v, wf_v, out_v, sem_in, sem_out):
    wid = lax.axis_index("s") * 2 + lax.axis_index("c")
    rows = wid * CPT
    pltpu.sync_copy(tblx_hbm.at[pl.ds(rows, CPT), :], tbl_v)

    def in_copies(b, par):
        vbase = b * VB
        return [
            pltpu.make_async_copy(
                c0_h.at[pl.ds(vbase, VB)], ci_v.at[par, 0], sem_in),
            pltpu.make_async_copy(
                c1_h.at[pl.ds(vbase, VB)], ci_v.at[par, 1], sem_in),
            pltpu.make_async_copy(
                w0_h.at[pl.ds(vbase, VB)], wf_v.at[par, 0], sem_in),
            pltpu.make_async_copy(
                w1_h.at[pl.ds(vbase, VB)], wf_v.at[par, 1], sem_in),
        ]

    def out_copy(b, par):
        return pltpu.make_async_copy(
            out_v.at[par],
            out_hbm.at[pl.ds(rows, CPT), pl.ds(b * VB, VB)], sem_out)

    for cp in in_copies(0, 0):
        cp.start()

    def blk_body(b, carry):
        par = lax.rem(b, 2)
        for cp in in_copies(b, par):
            cp.wait()

        @pl.when(b + 1 < NBLK)
        def _():
            for cp in in_copies(b + 1, 1 - par):
                cp.start()

        @pl.when(b >= 2)
        def _():
            out_copy(b - 2, par).wait()

        def g_body(g, carry2):
            accs = []
            for k in range(2):
                s = pl.ds((g * 2 + k) * 16, 16)
                c0 = ci_v[par, 0, s]
                c1 = ci_v[par, 1, s]
                w0 = wf_v[par, 0, s]
                w1 = wf_v[par, 1, s]
                for c in range(CPT):
                    cvec = jnp.full((16,), c, jnp.int32)
                    accs.append(w0 * plsc.load_gather(tbl_v, [cvec, c0])
                                + w1 * plsc.load_gather(tbl_v, [cvec, c1]))
            for k in range(2):
                s = pl.ds((g * 2 + k) * 16, 16)
                for c in range(CPT):
                    out_v[par, c, s] = accs[k * CPT + c]
            return carry2

        lax.fori_loop(0, VB // 32, g_body, 0)
        out_copy(b, par).start()
        return carry

    lax.fori_loop(0, NBLK, blk_body, 0)
    out_copy(NBLK - 2, lax.rem(NBLK - 2, 2)).wait()
    out_copy(NBLK - 1, lax.rem(NBLK - 1, 2)).wait()


def kernel(feature2d, depth_mapping_3d, conv_w, bn_gamma, bn_beta):
    f2d = feature2d.reshape(C_IN, NPIX)
    idx = depth_mapping_3d.reshape(N_VOX).astype(jnp.int32)
    tbl = pl.pallas_call(
        _stage1_body,
        out_shape=jax.ShapeDtypeStruct((FEAT, NPIX), jnp.float32),
    )(conv_w, f2d, bn_gamma.reshape(FEAT, 1), bn_beta.reshape(FEAT, 1))

    tblx = pl.pallas_call(
        _xup_body,
        out_shape=jax.ShapeDtypeStruct((FEAT, NXUP), jnp.float32),
    )(tbl)

    c0, c1, w0, w1 = pl.pallas_call(
        _prep_body,
        out_shape=[jax.ShapeDtypeStruct((N_VOX,), jnp.int32)] * 2
        + [jax.ShapeDtypeStruct((N_VOX,), jnp.float32)] * 2,
    )(idx)

    mesh = plsc.VectorSubcoreMesh(core_axis_name="c", subcore_axis_name="s")
    sc = functools.partial(
        pl.kernel,
        mesh=mesh,
        out_type=jax.ShapeDtypeStruct((FEAT, N_VOX), jnp.float32),
        scratch_types=[
            pltpu.VMEM((CPT, NXUP), jnp.float32),
            pltpu.VMEM((2, 2, VB), jnp.int32),
            pltpu.VMEM((2, 2, VB), jnp.float32),
            pltpu.VMEM((2, CPT, VB), jnp.float32),
            pltpu.SemaphoreType.DMA,
            pltpu.SemaphoreType.DMA,
        ],
        compiler_params=pltpu.CompilerParams(
            use_tc_tiling_on_sc=False, needs_layout_passes=False),
    )(_sc_gather)
    out = sc(tblx, c0, c1, w0, w1)
    return out.reshape(1, FEAT, 60, 36, 60)


# speedup vs baseline: 1.0653x; 1.0018x over previous
import functools

import jax
import jax.numpy as jnp
from jax import lax
from jax.experimental import pallas as pl
from jax.experimental.pallas import tpu as pltpu
from jax.experimental.pallas import tpu_sc as plsc

B, C_IN, H, W = 1, 2048, 15, 20
FEAT = 512
SCALE = 16
OH, OW = H * SCALE, W * SCALE
HW = OH * OW
NPIX = H * W
NXUP = H * OW
N_VOX = 60 * 36 * 60

NW = 32
CPT = FEAT // NW
VB = 576
NBLK = N_VOX // VB


def _stage1_body(w_ref, f2d_ref, gamma_ref, beta_ref, out_ref):
    x = jnp.dot(w_ref[...], f2d_ref[...], preferred_element_type=jnp.float32)
    mean = jnp.mean(x, axis=1, keepdims=True)
    var = jnp.mean(x * x, axis=1, keepdims=True) - mean * mean
    x = (x - mean) * jax.lax.rsqrt(var + 1e-5)
    x = x * gamma_ref[...] + beta_ref[...]
    tbl = jnp.maximum(x, 0.0)
    oxi = lax.broadcasted_iota(jnp.int32, (W, OW), 1)
    j = lax.broadcasted_iota(jnp.int32, (W, OW), 0)
    fx = oxi.astype(jnp.float32) * (float(W - 1) / (OW - 1))
    x0 = jnp.floor(fx)
    dx = fx - x0
    x0i = x0.astype(jnp.int32)
    x1i = jnp.minimum(x0i + 1, W - 1)
    wx = (jnp.where(j == x0i, 1.0 - dx, 0.0)
          + jnp.where(j == x1i, dx, 0.0))
    for y in range(H):
        out_ref[:, y * OW:(y + 1) * OW] = jnp.dot(
            tbl[:, y * W:(y + 1) * W], wx,
            preferred_element_type=jnp.float32)


def _prep_body(idx_ref, c0_ref, c1_ref, w0_ref, w1_ref):
    v = idx_ref[...]
    valid = v < HW
    vc = jnp.where(valid, v, 0)
    py = vc // OW
    px = vc - py * OW
    fy = py.astype(jnp.float32) * (float(H - 1) / (OH - 1))
    y0 = jnp.floor(fy)
    dy = fy - y0
    y0i = y0.astype(jnp.int32)
    y1i = jnp.minimum(y0i + 1, H - 1)
    vf = jnp.where(valid, 1.0, 0.0)
    c0_ref[...] = y0i * OW + px
    c1_ref[...] = y1i * OW + px
    w0_ref[...] = (1.0 - dy) * vf
    w1_ref[...] = dy * vf


def _sc_gather(tblx_hbm, c0_h, c1_h, w0_h, w1_h, out_hbm,
               tbl_v, ci_v, wf_v, out_v, sem_in, sem_out):
    wid = lax.axis_index("s") * 2 + lax.axis_index("c")
    rows = wid * CPT
    pltpu.sync_copy(tblx_hbm.at[pl.ds(rows, CPT), :], tbl_v)

    def in_copies(b, par):
        vbase = b * VB
        return [
            pltpu.make_async_copy(
                c0_h.at[pl.ds(vbase, VB)], ci_v.at[par, 0], sem_in),
            pltpu.make_async_copy(
                c1_h.at[pl.ds(vbase, VB)], ci_v.at[par, 1], sem_in),
            pltpu.make_async_copy(
                w0_h.at[pl.ds(vbase, VB)], wf_v.at[par, 0], sem_in),
            pltpu.make_async_copy(
                w1_h.at[pl.ds(vbase, VB)], wf_v.at[par, 1], sem_in),
        ]

    def out_copy(b, par):
        return pltpu.make_async_copy(
            out_v.at[par],
            out_hbm.at[pl.ds(rows, CPT), pl.ds(b * VB, VB)], sem_out)

    for cp in in_copies(0, 0):
        cp.start()

    def blk_body(b, carry):
        par = lax.rem(b, 2)
        for cp in in_copies(b, par):
            cp.wait()

        @pl.when(b + 1 < NBLK)
        def _():
            for cp in in_copies(b + 1, 1 - par):
                cp.start()

        @pl.when(b >= 2)
        def _():
            out_copy(b - 2, par).wait()

        def g_body(g, carry2):
            accs = []
            for k in range(2):
                s = pl.ds((g * 2 + k) * 16, 16)
                c0 = ci_v[par, 0, s]
                c1 = ci_v[par, 1, s]
                w0 = wf_v[par, 0, s]
                w1 = wf_v[par, 1, s]
                for c in range(CPT):
                    cvec = jnp.full((16,), c, jnp.int32)
                    accs.append(w0 * plsc.load_gather(tbl_v, [cvec, c0])
                                + w1 * plsc.load_gather(tbl_v, [cvec, c1]))
            for k in range(2):
                s = pl.ds((g * 2 + k) * 16, 16)
                for c in range(CPT):
                    out_v[par, c, s] = accs[k * CPT + c]
            return carry2

        lax.fori_loop(0, VB // 32, g_body, 0)
        out_copy(b, par).start()
        return carry

    lax.fori_loop(0, NBLK, blk_body, 0)
    out_copy(NBLK - 2, lax.rem(NBLK - 2, 2)).wait()
    out_copy(NBLK - 1, lax.rem(NBLK - 1, 2)).wait()


def kernel(feature2d, depth_mapping_3d, conv_w, bn_gamma, bn_beta):
    f2d = feature2d.reshape(C_IN, NPIX)
    idx = depth_mapping_3d.reshape(N_VOX).astype(jnp.int32)
    tblx = pl.pallas_call(
        _stage1_body,
        out_shape=jax.ShapeDtypeStruct((FEAT, NXUP), jnp.float32),
    )(conv_w, f2d, bn_gamma.reshape(FEAT, 1), bn_beta.reshape(FEAT, 1))

    c0, c1, w0, w1 = pl.pallas_call(
        _prep_body,
        out_shape=[jax.ShapeDtypeStruct((N_VOX,), jnp.int32)] * 2
        + [jax.ShapeDtypeStruct((N_VOX,), jnp.float32)] * 2,
    )(idx)

    mesh = plsc.VectorSubcoreMesh(core_axis_name="c", subcore_axis_name="s")
    sc = functools.partial(
        pl.kernel,
        mesh=mesh,
        out_type=jax.ShapeDtypeStruct((FEAT, N_VOX), jnp.float32),
        scratch_types=[
            pltpu.VMEM((CPT, NXUP), jnp.float32),
            pltpu.VMEM((2, 2, VB), jnp.int32),
            pltpu.VMEM((2, 2, VB), jnp.float32),
            pltpu.VMEM((2, CPT, VB), jnp.float32),
            pltpu.SemaphoreType.DMA,
            pltpu.SemaphoreType.DMA,
        ],
        compiler_params=pltpu.CompilerParams(
            use_tc_tiling_on_sc=False, needs_layout_passes=False),
    )(_sc_gather)
    out = sc(tblx, c0, c1, w0, w1)
    return out.reshape(1, FEAT, 60, 36, 60)
